# Initial kernel scaffold; baseline (speedup 1.0000x reference)
import jax
import jax.numpy as jnp
from jax.experimental import pallas as pl


def kernel(feat, edge_index, b, W_rel1, b_rel1, W_root1, W_rel2, b_rel2, W_root2, W1, bb1, W2, bb2, W3, bb3):
    raise NotImplementedError("write your pallas kernel here")


# speedup vs baseline: 25.6582x; 25.6582x over previous
import functools

import jax
import jax.numpy as jnp
from jax import lax
from jax.experimental import pallas as pl
from jax.experimental.pallas import tpu as pltpu
from jax.experimental.pallas import tpu_sc as plsc

N_NODES = 100000
E_EDGES = 1600000
N_GRAPHS = 64
BLK = 2048
N_PAD = 49 * BLK
NSUB = 16
STRIPE = N_PAD // NSUB
C1 = 2000
C2 = 1000

_sc_mesh = plsc.VectorSubcoreMesh(core_axis_name="c", subcore_axis_name="s")


@functools.partial(
    pl.kernel,
    out_type=jax.ShapeDtypeStruct((2, N_PAD), jnp.float32),
    mesh=_sc_mesh,
    scratch_types=[
        pltpu.VMEM((C1,), jnp.int32),
        pltpu.VMEM((C1,), jnp.int32),
        pltpu.VMEM((C1,), jnp.float32),
        pltpu.VMEM_SHARED((N_PAD,), jnp.float32),
        pltpu.VMEM_SHARED((N_PAD,), jnp.float32),
        pltpu.SemaphoreType.DMA,
    ],
    compiler_params=pltpu.CompilerParams(use_tc_tiling_on_sc=False),
)
def _sc1(feat_hbm, src_hbm, dst_hbm, zeros_hbm, out_hbm,
         sbuf, dbuf, vbuf, feat_sh, table, gsem):
    cid = lax.axis_index("c")
    sid = lax.axis_index("s")
    wid = cid * NSUB + sid
    pltpu.sync_copy(feat_hbm.at[pl.ds(sid * STRIPE, STRIPE)],
                    feat_sh.at[pl.ds(sid * STRIPE, STRIPE)])
    pltpu.sync_copy(zeros_hbm.at[pl.ds(sid * STRIPE, STRIPE)],
                    table.at[pl.ds(sid * STRIPE, STRIPE)])
    plsc.subcore_barrier()

    per_tile = E_EDGES // 32
    base = wid * per_tile

    def chunk(i, carry):
        eb = base + i * C1
        pltpu.sync_copy(src_hbm.at[pl.ds(eb, C1)], sbuf)
        pltpu.sync_copy(dst_hbm.at[pl.ds(eb, C1)], dbuf)
        pltpu.async_copy(feat_sh.at[sbuf], vbuf, gsem).wait()
        pltpu.sync_copy(vbuf, table.at[dbuf], add=True)
        return carry

    lax.fori_loop(0, per_tile // C1, chunk, 0)
    plsc.subcore_barrier()
    pltpu.sync_copy(table.at[pl.ds(sid * STRIPE, STRIPE)],
                    out_hbm.at[cid, pl.ds(sid * STRIPE, STRIPE)])


@functools.partial(
    pl.kernel,
    out_type=jax.ShapeDtypeStruct((2, N_PAD, 16), jnp.float32),
    mesh=_sc_mesh,
    scratch_types=[
        pltpu.VMEM((C2,), jnp.int32),
        pltpu.VMEM((C2,), jnp.int32),
        pltpu.VMEM((C2, 16), jnp.float32),
        pltpu.VMEM_SHARED((N_PAD, 16), jnp.float32),
        pltpu.SemaphoreType.DMA,
    ],
    compiler_params=pltpu.CompilerParams(use_tc_tiling_on_sc=False),
)
def _sc2(g_hbm, r_hbm, src2_hbm, dst_hbm, out_hbm,
         sbuf, dbuf, rows, table, gsem):
    cid = lax.axis_index("c")
    sid = lax.axis_index("s")
    pltpu.sync_copy(r_hbm.at[pl.ds(cid * N_PAD + sid * STRIPE, STRIPE)],
                    table.at[pl.ds(sid * STRIPE, STRIPE)])
    plsc.subcore_barrier()

    per_tile = E_EDGES // NSUB
    base = sid * per_tile

    def chunk(i, carry):
        eb = base + i * C2
        pltpu.sync_copy(src2_hbm.at[pl.ds(cid * E_EDGES + eb, C2)], sbuf)
        pltpu.sync_copy(dst_hbm.at[pl.ds(eb, C2)], dbuf)
        pltpu.async_copy(g_hbm.at[sbuf], rows, gsem).wait()
        pltpu.sync_copy(rows, table.at[dbuf], add=True)
        return carry

    lax.fori_loop(0, per_tile // C2, chunk, 0)
    plsc.subcore_barrier()
    pltpu.sync_copy(table.at[pl.ds(sid * STRIPE, STRIPE)],
                    out_hbm.at[cid, pl.ds(sid * STRIPE, STRIPE)])


def _tc1_body(agg_ref, feat_ref, wr1_ref, br1_ref, wq1_ref, wr2_ref, br2_ref,
              wq2_ref, g_ref, r_ref):
    a = agg_ref[0, :] + agg_ref[1, :]
    f = feat_ref[:]
    h1 = jnp.maximum(
        a[:, None] * wr1_ref[0, :][None, :]
        + f[:, None] * wq1_ref[0, :][None, :]
        + br1_ref[:][None, :], 0.0)
    g = jnp.dot(h1, wr2_ref[...], preferred_element_type=jnp.float32)
    r = jnp.dot(h1, wq2_ref[...], preferred_element_type=jnp.float32)
    r = r + br2_ref[:][None, :]
    g_ref[0] = g[:, :16]
    g_ref[1] = g[:, 16:]
    r_ref[0] = r[:, :16]
    r_ref[1] = r[:, 16:]


_tc1 = pl.pallas_call(
    _tc1_body,
    grid=(N_PAD // BLK,),
    in_specs=[
        pl.BlockSpec((2, BLK), lambda i: (0, i)),
        pl.BlockSpec((BLK,), lambda i: (i,)),
        pl.BlockSpec((1, 64), lambda i: (0, 0)),
        pl.BlockSpec((64,), lambda i: (0,)),
        pl.BlockSpec((1, 64), lambda i: (0, 0)),
        pl.BlockSpec((64, 32), lambda i: (0, 0)),
        pl.BlockSpec((32,), lambda i: (0,)),
        pl.BlockSpec((64, 32), lambda i: (0, 0)),
    ],
    out_specs=[
        pl.BlockSpec((2, BLK, 16), lambda i: (0, i, 0)),
        pl.BlockSpec((2, BLK, 16), lambda i: (0, i, 0)),
    ],
    out_shape=[
        jax.ShapeDtypeStruct((2, N_PAD, 16), jnp.float32),
        jax.ShapeDtypeStruct((2, N_PAD, 16), jnp.float32),
    ],
)


def _tc2_body(o_ref, b_ref, w1_ref, bb1_ref, w2_ref, bb2_ref, w3_ref, bb3_ref,
              out_ref, sacc, cacc):
    i = pl.program_id(0)

    @pl.when(i == 0)
    def _():
        sacc[...] = jnp.zeros_like(sacc)
        cacc[...] = jnp.zeros_like(cacc)

    h2 = jnp.concatenate([o_ref[0], o_ref[1]], axis=1)
    h2 = jnp.maximum(h2, 0.0)
    z = jnp.maximum(jnp.dot(h2, w1_ref[...],
                            preferred_element_type=jnp.float32)
                    + bb1_ref[:][None, :], 0.0)
    z = jnp.maximum(jnp.dot(z, w2_ref[...],
                            preferred_element_type=jnp.float32)
                    + bb2_ref[:][None, :], 0.0)
    y = jnp.dot(z, w3_ref[...],
                preferred_element_type=jnp.float32) + bb3_ref[:][None, :]
    bb = b_ref[:]
    gids = lax.broadcasted_iota(jnp.int32, (N_GRAPHS, BLK), 0)
    oh = (bb[None, :] == gids).astype(jnp.float32)
    sacc[...] += jnp.dot(oh, y, preferred_element_type=jnp.float32)
    cacc[...] += jnp.sum(oh, axis=1, keepdims=True)

    @pl.when(i == pl.num_programs(0) - 1)
    def _():
        pooled = sacc[...] / jnp.maximum(cacc[...], 1.0)
        out_ref[...] = jax.nn.sigmoid(pooled[:, 0])


_tc2 = pl.pallas_call(
    _tc2_body,
    grid=(N_PAD // BLK,),
    in_specs=[
        pl.BlockSpec((2, BLK, 16), lambda i: (0, i, 0)),
        pl.BlockSpec((BLK,), lambda i: (i,)),
        pl.BlockSpec((32, 16), lambda i: (0, 0)),
        pl.BlockSpec((16,), lambda i: (0,)),
        pl.BlockSpec((16, 8), lambda i: (0, 0)),
        pl.BlockSpec((8,), lambda i: (0,)),
        pl.BlockSpec((8, 1), lambda i: (0, 0)),
        pl.BlockSpec((1,), lambda i: (0,)),
    ],
    out_specs=pl.BlockSpec((N_GRAPHS,), lambda i: (0,)),
    out_shape=jax.ShapeDtypeStruct((N_GRAPHS,), jnp.float32),
    scratch_shapes=[
        pltpu.VMEM((N_GRAPHS, 1), jnp.float32),
        pltpu.VMEM((N_GRAPHS, 1), jnp.float32),
    ],
)


def kernel(feat, edge_index, b, W_rel1, b_rel1, W_root1, W_rel2, b_rel2,
           W_root2, W1, bb1, W2, bb2, W3, bb3):
    src = edge_index[0]
    dst = edge_index[1]
    feat_p = jnp.zeros((N_PAD,), jnp.float32).at[:N_NODES].set(feat[:, 0])
    b_p = jnp.full((N_PAD,), N_GRAPHS, jnp.int32).at[:N_NODES].set(b)
    src2 = jnp.concatenate([src, src + N_PAD])
    zeros_n = jnp.zeros((N_PAD,), jnp.float32)

    agg1 = _sc1(feat_p, src, dst, zeros_n)
    g3, r3 = _tc1(agg1, feat_p, W_rel1, b_rel1, W_root1,
                  W_rel2, b_rel2, W_root2)
    g2 = g3.reshape(2 * N_PAD, 16)
    r2 = r3.reshape(2 * N_PAD, 16)
    o = _sc2(g2, r2, src2, dst)
    return _tc2(o, b_p, W1, bb1, W2, bb2, W3, bb3)


# speedup vs baseline: 29.5141x; 1.1503x over previous
import functools

import jax
import jax.numpy as jnp
from jax import lax
from jax.experimental import pallas as pl
from jax.experimental.pallas import tpu as pltpu
from jax.experimental.pallas import tpu_sc as plsc

N_NODES = 100000
E_EDGES = 1600000
N_GRAPHS = 64
BLK = 2048
N_PAD = 49 * BLK
NSUB = 16
STRIPE = N_PAD // NSUB
C1 = 1000
C2 = 400

_sc_mesh = plsc.VectorSubcoreMesh(core_axis_name="c", subcore_axis_name="s")
_sc_params = pltpu.CompilerParams(use_tc_tiling_on_sc=False)


@functools.partial(
    pl.kernel,
    out_type=jax.ShapeDtypeStruct((2, N_PAD), jnp.float32),
    mesh=_sc_mesh,
    scratch_types=[
        pltpu.VMEM((2, C1), jnp.int32),
        pltpu.VMEM((2, C1), jnp.int32),
        pltpu.VMEM((2, C1), jnp.float32),
        pltpu.VMEM_SHARED((N_PAD,), jnp.float32),
        pltpu.VMEM_SHARED((N_PAD,), jnp.float32),
        pltpu.SemaphoreType.DMA,
        pltpu.SemaphoreType.DMA,
        pltpu.SemaphoreType.DMA,
        pltpu.SemaphoreType.DMA,
    ],
    compiler_params=_sc_params,
)
def _sc1(feat_hbm, src_hbm, dst_hbm, zeros_hbm, out_hbm,
         sbuf, dbuf, vbuf, feat_sh, table, gsem0, gsem1, isem0, isem1):
    cid = lax.axis_index("c")
    sid = lax.axis_index("s")
    wid = cid * NSUB + sid
    pltpu.sync_copy(feat_hbm.at[pl.ds(sid * STRIPE, STRIPE)],
                    feat_sh.at[pl.ds(sid * STRIPE, STRIPE)])
    pltpu.sync_copy(zeros_hbm.at[pl.ds(sid * STRIPE, STRIPE)],
                    table.at[pl.ds(sid * STRIPE, STRIPE)])
    plsc.subcore_barrier()

    per_tile = E_EDGES // 32
    base = wid * per_tile
    nch = per_tile // C1
    gsems = (gsem0, gsem1)
    isems = (isem0, isem1)

    def fire_idx(ch, b):
        eb = base + ch * C1
        pltpu.async_copy(src_hbm.at[pl.ds(eb, C1)], sbuf.at[b], isems[b])
        pltpu.async_copy(dst_hbm.at[pl.ds(eb, C1)], dbuf.at[b], isems[b])

    def wait_idx(b):
        pltpu.make_async_copy(src_hbm.at[pl.ds(0, C1)], sbuf.at[b],
                              isems[b]).wait()
        pltpu.make_async_copy(dst_hbm.at[pl.ds(0, C1)], dbuf.at[b],
                              isems[b]).wait()

    def fire_gather(b):
        pltpu.async_copy(feat_sh.at[sbuf.at[b]], vbuf.at[b], gsems[b])

    def wait_gather(b):
        pltpu.make_async_copy(feat_hbm.at[pl.ds(0, C1)], vbuf.at[b],
                              gsems[b]).wait()

    fire_idx(0, 0)
    wait_idx(0)
    fire_gather(0)
    fire_idx(1, 1)

    def pair(i2, carry):
        for b in (0, 1):
            ch = 2 * i2 + b
            nb = 1 - b

            @pl.when(ch + 1 < nch)
            def _():
                wait_idx(nb)
                fire_gather(nb)

            wait_gather(b)
            pltpu.sync_copy(vbuf.at[b], table.at[dbuf.at[b]], add=True)

            @pl.when(ch + 2 < nch)
            def _():
                fire_idx(ch + 2, b)
        return carry

    lax.fori_loop(0, nch // 2, pair, 0)
    plsc.subcore_barrier()
    pltpu.sync_copy(table.at[pl.ds(sid * STRIPE, STRIPE)],
                    out_hbm.at[cid, pl.ds(sid * STRIPE, STRIPE)])


@functools.partial(
    pl.kernel,
    out_type=(jax.ShapeDtypeStruct((N_PAD, 16), jnp.float32),
              jax.ShapeDtypeStruct((N_PAD, 16), jnp.float32)),
    mesh=_sc_mesh,
    scratch_types=[
        pltpu.VMEM((2, C2), jnp.int32),
        pltpu.VMEM((2, C2), jnp.int32),
        pltpu.VMEM((2, C2, 16), jnp.float32),
        pltpu.VMEM_SHARED((N_PAD, 16), jnp.float32),
        pltpu.SemaphoreType.DMA,
        pltpu.SemaphoreType.DMA,
        pltpu.SemaphoreType.DMA,
        pltpu.SemaphoreType.DMA,
    ],
    compiler_params=_sc_params,
)
def _sc2(g0_hbm, g1_hbm, r0_hbm, r1_hbm, src_hbm, dst_hbm, o0_hbm, o1_hbm,
         sbuf, dbuf, rows, table, gsem0, gsem1, isem0, isem1):
    cid = lax.axis_index("c")
    sid = lax.axis_index("s")

    @pl.when(cid == 0)
    def _():
        pltpu.sync_copy(r0_hbm.at[pl.ds(sid * STRIPE, STRIPE)],
                        table.at[pl.ds(sid * STRIPE, STRIPE)])

    @pl.when(cid == 1)
    def _():
        pltpu.sync_copy(r1_hbm.at[pl.ds(sid * STRIPE, STRIPE)],
                        table.at[pl.ds(sid * STRIPE, STRIPE)])

    plsc.subcore_barrier()

    per_tile = E_EDGES // NSUB
    base = sid * per_tile
    nch = per_tile // C2
    gsems = (gsem0, gsem1)
    isems = (isem0, isem1)

    def fire_idx(ch, b):
        eb = base + ch * C2
        pltpu.async_copy(src_hbm.at[pl.ds(eb, C2)], sbuf.at[b], isems[b])
        pltpu.async_copy(dst_hbm.at[pl.ds(eb, C2)], dbuf.at[b], isems[b])

    def wait_idx(b):
        pltpu.make_async_copy(src_hbm.at[pl.ds(0, C2)], sbuf.at[b],
                              isems[b]).wait()
        pltpu.make_async_copy(dst_hbm.at[pl.ds(0, C2)], dbuf.at[b],
                              isems[b]).wait()

    def fire_gather(b):
        @pl.when(cid == 0)
        def _():
            pltpu.async_copy(g0_hbm.at[sbuf.at[b]], rows.at[b], gsems[b])

        @pl.when(cid == 1)
        def _():
            pltpu.async_copy(g1_hbm.at[sbuf.at[b]], rows.at[b], gsems[b])

    def wait_gather(b):
        pltpu.make_async_copy(g0_hbm.at[sbuf.at[b]], rows.at[b],
                              gsems[b]).wait()

    fire_idx(0, 0)
    wait_idx(0)
    fire_gather(0)
    fire_idx(1, 1)

    def pair(i2, carry):
        for b in (0, 1):
            ch = 2 * i2 + b
            nb = 1 - b

            @pl.when(ch + 1 < nch)
            def _():
                wait_idx(nb)
                fire_gather(nb)

            wait_gather(b)
            pltpu.sync_copy(rows.at[b], table.at[dbuf.at[b]], add=True)

            @pl.when(ch + 2 < nch)
            def _():
                fire_idx(ch + 2, b)
        return carry

    lax.fori_loop(0, nch // 2, pair, 0)
    plsc.subcore_barrier()

    @pl.when(cid == 0)
    def _():
        pltpu.sync_copy(table.at[pl.ds(sid * STRIPE, STRIPE)],
                        o0_hbm.at[pl.ds(sid * STRIPE, STRIPE)])

    @pl.when(cid == 1)
    def _():
        pltpu.sync_copy(table.at[pl.ds(sid * STRIPE, STRIPE)],
                        o1_hbm.at[pl.ds(sid * STRIPE, STRIPE)])


def _tc1_body(agg_ref, feat_ref, wr1_ref, br1_ref, wq1_ref, wr2_ref, br2_ref,
              wq2_ref, g0_ref, g1_ref, r0_ref, r1_ref):
    a = agg_ref[0, :] + agg_ref[1, :]
    f = feat_ref[:]
    h1 = jnp.maximum(
        a[:, None] * wr1_ref[0, :][None, :]
        + f[:, None] * wq1_ref[0, :][None, :]
        + br1_ref[:][None, :], 0.0)
    g = jnp.dot(h1, wr2_ref[...], preferred_element_type=jnp.float32)
    r = jnp.dot(h1, wq2_ref[...], preferred_element_type=jnp.float32)
    r = r + br2_ref[:][None, :]
    g0_ref[...] = g[:, :16]
    g1_ref[...] = g[:, 16:]
    r0_ref[...] = r[:, :16]
    r1_ref[...] = r[:, 16:]


_tc1 = pl.pallas_call(
    _tc1_body,
    grid=(N_PAD // BLK,),
    in_specs=[
        pl.BlockSpec((2, BLK), lambda i: (0, i)),
        pl.BlockSpec((BLK,), lambda i: (i,)),
        pl.BlockSpec((1, 64), lambda i: (0, 0)),
        pl.BlockSpec((64,), lambda i: (0,)),
        pl.BlockSpec((1, 64), lambda i: (0, 0)),
        pl.BlockSpec((64, 32), lambda i: (0, 0)),
        pl.BlockSpec((32,), lambda i: (0,)),
        pl.BlockSpec((64, 32), lambda i: (0, 0)),
    ],
    out_specs=[
        pl.BlockSpec((BLK, 16), lambda i: (i, 0)),
        pl.BlockSpec((BLK, 16), lambda i: (i, 0)),
        pl.BlockSpec((BLK, 16), lambda i: (i, 0)),
        pl.BlockSpec((BLK, 16), lambda i: (i, 0)),
    ],
    out_shape=[jax.ShapeDtypeStruct((N_PAD, 16), jnp.float32)] * 4,
)


def _tc2_body(o0_ref, o1_ref, b_ref, w1_ref, bb1_ref, w2_ref, bb2_ref,
              w3_ref, bb3_ref, out_ref, sacc, cacc):
    i = pl.program_id(0)

    @pl.when(i == 0)
    def _():
        sacc[...] = jnp.zeros_like(sacc)
        cacc[...] = jnp.zeros_like(cacc)

    h2 = jnp.concatenate([o0_ref[...], o1_ref[...]], axis=1)
    h2 = jnp.maximum(h2, 0.0)
    z = jnp.maximum(jnp.dot(h2, w1_ref[...],
                            preferred_element_type=jnp.float32)
                    + bb1_ref[:][None, :], 0.0)
    z = jnp.maximum(jnp.dot(z, w2_ref[...],
                            preferred_element_type=jnp.float32)
                    + bb2_ref[:][None, :], 0.0)
    y = jnp.dot(z, w3_ref[...],
                preferred_element_type=jnp.float32) + bb3_ref[:][None, :]
    bb = b_ref[:]
    gids = lax.broadcasted_iota(jnp.int32, (N_GRAPHS, BLK), 0)
    oh = (bb[None, :] == gids).astype(jnp.float32)
    sacc[...] += jnp.dot(oh, y, preferred_element_type=jnp.float32)
    cacc[...] += jnp.sum(oh, axis=1, keepdims=True)

    @pl.when(i == pl.num_programs(0) - 1)
    def _():
        pooled = sacc[...] / jnp.maximum(cacc[...], 1.0)
        out_ref[...] = jax.nn.sigmoid(pooled[:, 0])


_tc2 = pl.pallas_call(
    _tc2_body,
    grid=(N_PAD // BLK,),
    in_specs=[
        pl.BlockSpec((BLK, 16), lambda i: (i, 0)),
        pl.BlockSpec((BLK, 16), lambda i: (i, 0)),
        pl.BlockSpec((BLK,), lambda i: (i,)),
        pl.BlockSpec((32, 16), lambda i: (0, 0)),
        pl.BlockSpec((16,), lambda i: (0,)),
        pl.BlockSpec((16, 8), lambda i: (0, 0)),
        pl.BlockSpec((8,), lambda i: (0,)),
        pl.BlockSpec((8, 1), lambda i: (0, 0)),
        pl.BlockSpec((1,), lambda i: (0,)),
    ],
    out_specs=pl.BlockSpec((N_GRAPHS,), lambda i: (0,)),
    out_shape=jax.ShapeDtypeStruct((N_GRAPHS,), jnp.float32),
    scratch_shapes=[
        pltpu.VMEM((N_GRAPHS, 1), jnp.float32),
        pltpu.VMEM((N_GRAPHS, 1), jnp.float32),
    ],
)


def kernel(feat, edge_index, b, W_rel1, b_rel1, W_root1, W_rel2, b_rel2,
           W_root2, W1, bb1, W2, bb2, W3, bb3):
    src = edge_index[0]
    dst = edge_index[1]
    feat_p = jnp.zeros((N_PAD,), jnp.float32).at[:N_NODES].set(feat[:, 0])
    b_p = jnp.full((N_PAD,), N_GRAPHS, jnp.int32).at[:N_NODES].set(b)
    zeros_n = jnp.zeros((N_PAD,), jnp.float32)

    agg1 = _sc1(feat_p, src, dst, zeros_n)
    g0, g1, r0, r1 = _tc1(agg1, feat_p, W_rel1, b_rel1, W_root1,
                          W_rel2, b_rel2, W_root2)
    o0, o1 = _sc2(g0, g1, r0, r1, src, dst)
    return _tc2(o0, o1, b_p, W1, bb1, W2, bb2, W3, bb3)


# speedup vs baseline: 34.9257x; 1.1834x over previous
import functools

import jax
import jax.numpy as jnp
from jax import lax
from jax.experimental import pallas as pl
from jax.experimental.pallas import tpu as pltpu
from jax.experimental.pallas import tpu_sc as plsc

N_NODES = 100000
E_EDGES = 1600000
N_GRAPHS = 64
BLK = 2048
N_PAD = 49 * BLK
NSUB = 16
STRIPE = N_PAD // NSUB
C1 = 1000
C2 = 200
NBUF = 4

_sc_mesh = plsc.VectorSubcoreMesh(core_axis_name="c", subcore_axis_name="s")
_sc_params = pltpu.CompilerParams(use_tc_tiling_on_sc=False)


@functools.partial(
    pl.kernel,
    out_type=jax.ShapeDtypeStruct((2, N_PAD), jnp.float32),
    mesh=_sc_mesh,
    scratch_types=[
        pltpu.VMEM((NBUF, C1), jnp.int32),
        pltpu.VMEM((NBUF, C1), jnp.int32),
        pltpu.VMEM((NBUF, C1), jnp.float32),
        pltpu.VMEM_SHARED((N_PAD,), jnp.float32),
        pltpu.VMEM_SHARED((N_PAD,), jnp.float32),
    ] + [pltpu.SemaphoreType.DMA] * (2 * NBUF),
    compiler_params=_sc_params,
)
def _sc1(feat_hbm, src_hbm, dst_hbm, zeros_hbm, out_hbm,
         sbuf, dbuf, vbuf, feat_sh, table, *sems):
    cid = lax.axis_index("c")
    sid = lax.axis_index("s")
    wid = cid * NSUB + sid
    pltpu.sync_copy(feat_hbm.at[pl.ds(sid * STRIPE, STRIPE)],
                    feat_sh.at[pl.ds(sid * STRIPE, STRIPE)])
    pltpu.sync_copy(zeros_hbm.at[pl.ds(sid * STRIPE, STRIPE)],
                    table.at[pl.ds(sid * STRIPE, STRIPE)])
    plsc.subcore_barrier()

    per_tile = E_EDGES // 32
    base = wid * per_tile
    nch = per_tile // C1
    gsems = sems[:NBUF]
    isems = sems[NBUF:]

    def fire_idx(ch, b):
        eb = base + ch * C1
        pltpu.async_copy(src_hbm.at[pl.ds(eb, C1)], sbuf.at[b], isems[b])
        pltpu.async_copy(dst_hbm.at[pl.ds(eb, C1)], dbuf.at[b], isems[b])

    def wait_idx(b):
        pltpu.make_async_copy(src_hbm.at[pl.ds(0, C1)], sbuf.at[b],
                              isems[b]).wait()
        pltpu.make_async_copy(dst_hbm.at[pl.ds(0, C1)], dbuf.at[b],
                              isems[b]).wait()

    def fire_gather(b):
        pltpu.async_copy(feat_sh.at[sbuf.at[b]], vbuf.at[b], gsems[b])

    def wait_gather(b):
        pltpu.make_async_copy(feat_hbm.at[pl.ds(0, C1)], vbuf.at[b],
                              gsems[b]).wait()

    for c in range(NBUF - 1):
        fire_idx(c, c)
        wait_idx(c)
        fire_gather(c)
    fire_idx(NBUF - 1, NBUF - 1)

    def quad(i4, carry):
        for b in range(NBUF):
            ch = NBUF * i4 + b

            @pl.when(ch < nch)
            def _():
                nb = (b + NBUF - 1) % NBUF

                @pl.when(ch + NBUF - 1 < nch)
                def _():
                    wait_idx(nb)
                    fire_gather(nb)

                wait_gather(b)
                pltpu.sync_copy(vbuf.at[b], table.at[dbuf.at[b]], add=True)

                @pl.when(ch + NBUF < nch)
                def _():
                    fire_idx(ch + NBUF, b)
        return carry

    lax.fori_loop(0, (nch + NBUF - 1) // NBUF, quad, 0)
    plsc.subcore_barrier()
    pltpu.sync_copy(table.at[pl.ds(sid * STRIPE, STRIPE)],
                    out_hbm.at[cid, pl.ds(sid * STRIPE, STRIPE)])


@functools.partial(
    pl.kernel,
    out_type=(jax.ShapeDtypeStruct((N_PAD, 16), jnp.float32),
              jax.ShapeDtypeStruct((N_PAD, 16), jnp.float32)),
    mesh=_sc_mesh,
    scratch_types=[
        pltpu.VMEM((NBUF, C2), jnp.int32),
        pltpu.VMEM((NBUF, C2), jnp.int32),
        pltpu.VMEM((NBUF, C2, 16), jnp.float32),
        pltpu.VMEM_SHARED((N_PAD, 16), jnp.float32),
    ] + [pltpu.SemaphoreType.DMA] * (2 * NBUF),
    compiler_params=_sc_params,
)
def _sc2(g0_hbm, g1_hbm, r0_hbm, r1_hbm, src_hbm, dst_hbm, o0_hbm, o1_hbm,
         sbuf, dbuf, rows, table, *sems):
    cid = lax.axis_index("c")
    sid = lax.axis_index("s")

    @pl.when(cid == 0)
    def _():
        pltpu.sync_copy(r0_hbm.at[pl.ds(sid * STRIPE, STRIPE)],
                        table.at[pl.ds(sid * STRIPE, STRIPE)])

    @pl.when(cid == 1)
    def _():
        pltpu.sync_copy(r1_hbm.at[pl.ds(sid * STRIPE, STRIPE)],
                        table.at[pl.ds(sid * STRIPE, STRIPE)])

    plsc.subcore_barrier()

    per_tile = E_EDGES // NSUB
    base = sid * per_tile
    nch = per_tile // C2
    gsems = sems[:NBUF]
    isems = sems[NBUF:]

    def fire_idx(ch, b):
        eb = base + ch * C2
        pltpu.async_copy(src_hbm.at[pl.ds(eb, C2)], sbuf.at[b], isems[b])
        pltpu.async_copy(dst_hbm.at[pl.ds(eb, C2)], dbuf.at[b], isems[b])

    def wait_idx(b):
        pltpu.make_async_copy(src_hbm.at[pl.ds(0, C2)], sbuf.at[b],
                              isems[b]).wait()
        pltpu.make_async_copy(dst_hbm.at[pl.ds(0, C2)], dbuf.at[b],
                              isems[b]).wait()

    def fire_gather(b):
        @pl.when(cid == 0)
        def _():
            pltpu.async_copy(g0_hbm.at[sbuf.at[b]], rows.at[b], gsems[b])

        @pl.when(cid == 1)
        def _():
            pltpu.async_copy(g1_hbm.at[sbuf.at[b]], rows.at[b], gsems[b])

    def wait_gather(b):
        pltpu.make_async_copy(g0_hbm.at[sbuf.at[b]], rows.at[b],
                              gsems[b]).wait()

    for c in range(NBUF - 1):
        fire_idx(c, c)
        wait_idx(c)
        fire_gather(c)
    fire_idx(NBUF - 1, NBUF - 1)

    def quad(i4, carry):
        for b in range(NBUF):
            ch = NBUF * i4 + b

            @pl.when(ch < nch)
            def _():
                nb = (b + NBUF - 1) % NBUF

                @pl.when(ch + NBUF - 1 < nch)
                def _():
                    wait_idx(nb)
                    fire_gather(nb)

                wait_gather(b)
                pltpu.sync_copy(rows.at[b], table.at[dbuf.at[b]], add=True)

                @pl.when(ch + NBUF < nch)
                def _():
                    fire_idx(ch + NBUF, b)
        return carry

    lax.fori_loop(0, (nch + NBUF - 1) // NBUF, quad, 0)
    plsc.subcore_barrier()

    @pl.when(cid == 0)
    def _():
        pltpu.sync_copy(table.at[pl.ds(sid * STRIPE, STRIPE)],
                        o0_hbm.at[pl.ds(sid * STRIPE, STRIPE)])

    @pl.when(cid == 1)
    def _():
        pltpu.sync_copy(table.at[pl.ds(sid * STRIPE, STRIPE)],
                        o1_hbm.at[pl.ds(sid * STRIPE, STRIPE)])


PBLK = BLK // 8


def _kron8(w, bi, bj):
    t = jnp.concatenate([w] * 8, axis=0)
    t = jnp.concatenate([t] * 8, axis=1)
    ii = lax.broadcasted_iota(jnp.int32, (8 * bi, 8 * bj), 0) // bi
    jj = lax.broadcasted_iota(jnp.int32, (8 * bi, 8 * bj), 1) // bj
    return t * (ii == jj).astype(jnp.float32)


def _tile8(v):
    return jnp.concatenate([v] * 8)


def _tc1_body(agg_ref, feat_ref, wr1_ref, br1_ref, wq1_ref, wr2_ref, br2_ref,
              wq2_ref, g0_ref, g1_ref, r0_ref, r1_ref):
    a_p = agg_ref[0] + agg_ref[1]
    f_p = feat_ref[...]
    rsel = (lax.broadcasted_iota(jnp.int32, (8, 512), 0)
            == lax.broadcasted_iota(jnp.int32, (8, 512), 1) // 64
            ).astype(jnp.float32)
    a_e = jnp.dot(a_p, rsel, preferred_element_type=jnp.float32)
    f_e = jnp.dot(f_p, rsel, preferred_element_type=jnp.float32)
    wr1 = _tile8(wr1_ref[0, :])
    wq1 = _tile8(wq1_ref[0, :])
    br1 = _tile8(br1_ref[:])
    h1 = jnp.maximum(a_e * wr1[None, :] + f_e * wq1[None, :]
                     + br1[None, :], 0.0)
    w2 = wr2_ref[...]
    q2 = wq2_ref[...]
    g0_ref[...] = jnp.dot(h1, _kron8(w2[:, :16], 64, 16),
                          preferred_element_type=jnp.float32)
    g1_ref[...] = jnp.dot(h1, _kron8(w2[:, 16:], 64, 16),
                          preferred_element_type=jnp.float32)
    b2a = _tile8(br2_ref[:16])
    b2b = _tile8(br2_ref[16:])
    r0_ref[...] = jnp.dot(h1, _kron8(q2[:, :16], 64, 16),
                          preferred_element_type=jnp.float32) + b2a[None, :]
    r1_ref[...] = jnp.dot(h1, _kron8(q2[:, 16:], 64, 16),
                          preferred_element_type=jnp.float32) + b2b[None, :]


_tc1 = pl.pallas_call(
    _tc1_body,
    grid=(N_PAD // BLK,),
    in_specs=[
        pl.BlockSpec((2, PBLK, 8), lambda i: (0, i, 0)),
        pl.BlockSpec((PBLK, 8), lambda i: (i, 0)),
        pl.BlockSpec((1, 64), lambda i: (0, 0)),
        pl.BlockSpec((64,), lambda i: (0,)),
        pl.BlockSpec((1, 64), lambda i: (0, 0)),
        pl.BlockSpec((64, 32), lambda i: (0, 0)),
        pl.BlockSpec((32,), lambda i: (0,)),
        pl.BlockSpec((64, 32), lambda i: (0, 0)),
    ],
    out_specs=[
        pl.BlockSpec((PBLK, 128), lambda i: (i, 0)),
        pl.BlockSpec((PBLK, 128), lambda i: (i, 0)),
        pl.BlockSpec((PBLK, 128), lambda i: (i, 0)),
        pl.BlockSpec((PBLK, 128), lambda i: (i, 0)),
    ],
    out_shape=[jax.ShapeDtypeStruct((N_PAD // 8, 128), jnp.float32)] * 4,
)


def _tc2_body(o0_ref, o1_ref, bt_ref, w1_ref, bb1_ref, w2_ref, bb2_ref,
              w3_ref, bb3_ref, out_ref, acc):
    i = pl.program_id(0)

    @pl.when(i == 0)
    def _():
        acc[...] = jnp.zeros_like(acc)

    h0 = jnp.maximum(o0_ref[...], 0.0)
    h1 = jnp.maximum(o1_ref[...], 0.0)
    w1 = w1_ref[...]
    z = jnp.maximum(
        jnp.dot(h0, _kron8(w1[:16, :], 16, 16),
                preferred_element_type=jnp.float32)
        + jnp.dot(h1, _kron8(w1[16:, :], 16, 16),
                  preferred_element_type=jnp.float32)
        + _tile8(bb1_ref[:])[None, :], 0.0)
    z = jnp.maximum(
        jnp.dot(z, _kron8(w2_ref[...], 16, 8),
                preferred_element_type=jnp.float32)
        + _tile8(bb2_ref[:])[None, :], 0.0)
    y = (jnp.dot(z, _kron8(w3_ref[...], 8, 1),
                 preferred_element_type=jnp.float32)
         + bb3_ref[0])
    ones = jnp.ones((PBLK, 1), jnp.float32)
    gcol = lax.broadcasted_iota(jnp.int32, (N_GRAPHS, PBLK), 0)
    for n in range(8):
        bn = bt_ref[n:n + 1, :]
        oh = (jnp.broadcast_to(bn, (N_GRAPHS, PBLK)) == gcol
              ).astype(jnp.float32)
        rhs = jnp.concatenate([y[:, n:n + 1], ones], axis=1)
        acc[...] += jnp.dot(oh, rhs, preferred_element_type=jnp.float32)

    @pl.when(i == pl.num_programs(0) - 1)
    def _():
        pooled = acc[:, 0] / jnp.maximum(acc[:, 1], 1.0)
        out_ref[...] = jax.nn.sigmoid(pooled)


_tc2 = pl.pallas_call(
    _tc2_body,
    grid=(N_PAD // BLK,),
    in_specs=[
        pl.BlockSpec((PBLK, 128), lambda i: (i, 0)),
        pl.BlockSpec((PBLK, 128), lambda i: (i, 0)),
        pl.BlockSpec((8, PBLK), lambda i: (0, i)),
        pl.BlockSpec((32, 16), lambda i: (0, 0)),
        pl.BlockSpec((16,), lambda i: (0,)),
        pl.BlockSpec((16, 8), lambda i: (0, 0)),
        pl.BlockSpec((8,), lambda i: (0,)),
        pl.BlockSpec((8, 1), lambda i: (0, 0)),
        pl.BlockSpec((1,), lambda i: (0,)),
    ],
    out_specs=pl.BlockSpec((N_GRAPHS,), lambda i: (0,)),
    out_shape=jax.ShapeDtypeStruct((N_GRAPHS,), jnp.float32),
    scratch_shapes=[
        pltpu.VMEM((N_GRAPHS, 2), jnp.float32),
    ],
)


def kernel(feat, edge_index, b, W_rel1, b_rel1, W_root1, W_rel2, b_rel2,
           W_root2, W1, bb1, W2, bb2, W3, bb3):
    src = edge_index[0]
    dst = edge_index[1]
    feat_p = jnp.zeros((N_PAD,), jnp.float32).at[:N_NODES].set(feat[:, 0])
    b_p = jnp.full((N_PAD,), N_GRAPHS, jnp.int32).at[:N_NODES].set(b)
    zeros_n = jnp.zeros((N_PAD,), jnp.float32)

    agg1 = _sc1(feat_p, src, dst, zeros_n)
    agg1_p = agg1.reshape(2, N_PAD // 8, 8)
    feat_pp = feat_p.reshape(N_PAD // 8, 8)
    b_pt = b_p.reshape(N_PAD // 8, 8).T
    g0, g1, r0, r1 = _tc1(agg1_p, feat_pp, W_rel1, b_rel1, W_root1,
                          W_rel2, b_rel2, W_root2)
    g0, g1, r0, r1 = (x.reshape(N_PAD, 16) for x in (g0, g1, r0, r1))
    o0, o1 = _sc2(g0, g1, r0, r1, src, dst)
    o0 = o0.reshape(N_PAD // 8, 128)
    o1 = o1.reshape(N_PAD // 8, 128)
    return _tc2(o0, o1, b_pt, W1, bb1, W2, bb2, W3, bb3)
